# baseline (speedup 1.0000x reference)
import functools

import jax
import jax.numpy as jnp
from jax import lax
from jax.experimental import pallas as pl
from jax.experimental.pallas import tpu as pltpu
from jax.experimental.pallas import tpu_sc as plsc

E = 320000
N = 10000
HID = 128
NRAD = 16
CHUNK = 80
NCH = E // CHUNK
NC = 2
NS = 16
NW = NC * NS
NDRAIN = N // CHUNK
L = 16


def _tc_rf(rbT, W_r1, b_r1, W_r2, b_r2):
    BLK = 2560

    def body(rbT_ref, wr1, br1, wr2, br2, rf_ref):
        x = lax.dot_general(rbT_ref[...], wr1[...],
                            (((0,), (0,)), ((), ())),
                            preferred_element_type=jnp.float32)
        t = jax.nn.silu(x + br1[...])
        rf_ref[...] = (jnp.dot(t, wr2[...], preferred_element_type=jnp.float32)
                       + br2[...]).astype(jnp.bfloat16)

    def full(shape):
        return pl.BlockSpec(shape, lambda i: tuple(0 for _ in shape))

    return pl.pallas_call(
        body,
        grid=(E // BLK,),
        in_specs=[
            pl.BlockSpec((NRAD, BLK), lambda i: (0, i)),
            full((NRAD, HID)), full((1, HID)), full((HID, HID)), full((1, HID)),
        ],
        out_specs=pl.BlockSpec((BLK, HID), lambda i: (i, 0)),
        out_shape=jax.ShapeDtypeStruct((E, HID), jnp.bfloat16),
    )(rbT, W_r1, b_r1, W_r2, b_r2)


EAW = 327680


def _tc_aw(sbT8, w8b):
    BLK = 4096

    def body(sb_ref, w8_ref, aw_ref):
        z = jnp.sum(sb_ref[...] * w8_ref[...], axis=0)
        aw_ref[...] = jax.nn.sigmoid(z)

    return pl.pallas_call(
        body,
        grid=(EAW // BLK,),
        in_specs=[
            pl.BlockSpec((8, BLK), lambda i: (0, i)),
            pl.BlockSpec((8, BLK), lambda i: (0, 0)),
        ],
        out_specs=pl.BlockSpec((BLK,), lambda i: (i,)),
        out_shape=jax.ShapeDtypeStruct((EAW,), jnp.float32),
    )(sbT8, w8b)


NITER = NCH // NW

_DN = lax.GatherDimensionNumbers(
    offset_dims=(), collapsed_slice_dims=(0,), start_index_map=(0,))


def _sc_mega(m, rf, aw, ikj, iji, dst, chunk0, nchunks):
    mesh = plsc.VectorSubcoreMesh(core_axis_name="c", subcore_axis_name="s")
    niter = nchunks // NW
    goff = chunk0 * CHUNK

    nbuf = 2
    SETLEN = 9
    scratch = []
    for _ in range(nbuf):
        scratch += [
            pltpu.VMEM((CHUNK,), jnp.int32),
            pltpu.VMEM((CHUNK,), jnp.int32),
            pltpu.VMEM((CHUNK,), jnp.int32),
            pltpu.VMEM((CHUNK, HID), jnp.float32),
            pltpu.VMEM((CHUNK, HID), jnp.bfloat16),
            pltpu.VMEM((CHUNK,), jnp.float32),
            pltpu.SemaphoreType.DMA,
            pltpu.SemaphoreType.DMA,
            pltpu.VMEM((CHUNK, HID), jnp.bfloat16),
        ]
    scratch.append(pltpu.VMEM_SHARED((N, HID), jnp.float32))

    @functools.partial(
        pl.kernel,
        out_type=(
            jax.ShapeDtypeStruct((nchunks * CHUNK, HID), jnp.bfloat16),
            jax.ShapeDtypeStruct((NC, N, HID), jnp.float32),
        ),
        mesh=mesh,
        scratch_types=scratch,
        compiler_params=pltpu.CompilerParams(use_tc_tiling_on_sc=False, needs_layout_passes=False),
    )
    def k(m_hbm, rf_hbm, aw_hbm, ikj_hbm, iji_hbm, dst_hbm,
          mf_out, agg_out, *bufs):
        sets = []
        for b in range(nbuf):
            sets.append(bufs[b * SETLEN:(b + 1) * SETLEN])
        agg_sh = bufs[nbuf * SETLEN]
        c = lax.axis_index("c")
        s = lax.axis_index("s")
        wid = s * NC + c

        zb = sets[0][3]

        def zrow(i, carry):
            r = i // (HID // L)
            q = i % (HID // L)
            zb[r, pl.ds(q * L, L)] = jnp.zeros((L,), jnp.float32)
            return carry

        lax.fori_loop(0, CHUNK * (HID // L), zrow, 0)
        for j in range((NDRAIN + NS - 1) // NS):
            t = j * NS + s

            @pl.when(t < NDRAIN)
            def _():
                pltpu.sync_copy(
                    zb, agg_sh.at[pl.ds(pl.multiple_of(t * CHUNK, CHUNK),
                                        CHUNK)])

        plsc.subcore_barrier()

        def issue_idx(i, st):
            base = pl.multiple_of(goff + (i * NW + wid) * CHUNK, CHUNK)
            a = pltpu.async_copy(ikj_hbm.at[pl.ds(base, CHUNK)], st[0], st[6])
            b = pltpu.async_copy(iji_hbm.at[pl.ds(base, CHUNK)], st[1], st[6])
            d = pltpu.async_copy(dst_hbm.at[pl.ds(base, CHUNK)], st[2], st[6])
            return a, b, d

        def issue_idx_drain(st):
            pltpu.make_async_copy(ikj_hbm.at[pl.ds(0, CHUNK)], st[0], st[6]).wait()
            pltpu.make_async_copy(iji_hbm.at[pl.ds(0, CHUNK)], st[1], st[6]).wait()
            pltpu.make_async_copy(dst_hbm.at[pl.ds(0, CHUNK)], st[2], st[6]).wait()

        def issue_gath(st):
            pltpu.async_copy(m_hbm.at[st[0]], st[3], st[7])
            pltpu.async_copy(rf_hbm.at[st[0]], st[4], st[7])
            pltpu.async_copy(aw_hbm.at[st[1]], st[5], st[7])

        def wait_gath(st):
            pltpu.make_async_copy(m_hbm.at[st[0]], st[3], st[7]).wait()
            pltpu.make_async_copy(rf_hbm.at[st[0]], st[4], st[7]).wait()
            pltpu.make_async_copy(aw_hbm.at[st[1]], st[5], st[7]).wait()

        def mul(st):
            mbuf, rfbuf, awbuf, pbuf = st[3], st[4], st[5], st[8]

            def mul_group(p, carry2):
                aw16 = awbuf[pl.ds(pl.multiple_of(p * L, L), L)]
                for l in range(L):
                    awr = lax.gather(
                        aw16, jnp.full((L, 1), l, jnp.int32), _DN,
                        slice_sizes=(1,),
                        mode=lax.GatherScatterMode.PROMISE_IN_BOUNDS)
                    r = p * L + l
                    for q in range(HID // (2 * L)):
                        ra, rb = plsc.unpack(
                            rfbuf[r, pl.ds(q * 2 * L, 2 * L)],
                            format=plsc.PackFormat.INTERLEAVED,
                            preferred_element_type=jnp.float32)
                        sla = pl.ds(q * 2 * L, L)
                        slb = pl.ds(q * 2 * L + L, L)
                        va = mbuf[r, sla] * ra * awr
                        vb = mbuf[r, slb] * rb * awr
                        mbuf[r, sla] = va
                        mbuf[r, slb] = vb
                        pbuf[r, pl.ds(q * 2 * L, 2 * L)] = plsc.pack(
                            va, vb, format=plsc.PackFormat.INTERLEAVED)
                return carry2

            lax.fori_loop(0, CHUNK // L, mul_group, 0)

        def consume(i, st):
            wait_gath(st)
            mul(st)
            base = pl.multiple_of((i * NW + wid) * CHUNK, CHUNK)
            pltpu.sync_copy(st[8], mf_out.at[pl.ds(base, CHUNK)])
            pltpu.sync_copy(st[3], agg_sh.at[st[2]], add=True)

        NITER = niter

        A, B = sets[0], sets[1]
        issue_idx(0, A)
        issue_idx_drain(A)
        issue_gath(A)
        issue_idx(1, B)

        def pair_body(p, carry):
            i0 = p * 2
            i1 = i0 + 1

            @pl.when(i1 < NITER)
            def _():
                issue_idx_drain(B)
                issue_gath(B)

            consume(i0, A)

            @pl.when(i0 + 2 < NITER)
            def _():
                issue_idx(i0 + 2, A)

            @pl.when(i1 < NITER)
            def _():
                @pl.when(i1 + 1 < NITER)
                def _():
                    issue_idx_drain(A)
                    issue_gath(A)

                consume(i1, B)

                @pl.when(i1 + 2 < NITER)
                def _():
                    issue_idx(i1 + 2, B)

            return carry

        lax.fori_loop(0, (NITER + 1) // 2, pair_body, 0)

        plsc.subcore_barrier()

        for j in range((NDRAIN + NS - 1) // NS):
            t = j * NS + s

            @pl.when(t < NDRAIN)
            def _():
                base = pl.multiple_of(t * CHUNK, CHUNK)
                pltpu.sync_copy(agg_sh.at[pl.ds(base, CHUNK)],
                                agg_out.at[c, pl.ds(base, CHUNK)])

    return k(m, rf, aw, ikj, iji, dst)


def _tc_edges(m, mf, W_o1, b_o1, W_o2, b_o2, W_o3, b_o3, blk0, mout_prev):
    BLK = 2560
    nblk = mf.shape[0] // BLK

    def body(*refs):
        m_ref, mf_ref, wo1, bo1, wo2, bo2, wo3, bo3 = refs[:8]
        mout_ref = refs[-1]
        silu = jax.nn.silu
        mfv = mf_ref[...].astype(jnp.float32)
        mn = silu(jnp.dot(mfv, wo1[...], preferred_element_type=jnp.float32)
                  + bo1[...])
        mn = mn + silu(jnp.dot(mfv, wo2[...], preferred_element_type=jnp.float32)
                       + bo2[...])
        mn = mn + silu(jnp.dot(mfv, wo3[...], preferred_element_type=jnp.float32)
                       + bo3[...])
        mout_ref[...] = m_ref[...] + mn

    def full(shape):
        return pl.BlockSpec(shape, lambda i: tuple(0 for _ in shape))

    in_specs = [
        pl.BlockSpec((BLK, HID), lambda i: (i + blk0, 0)),
        pl.BlockSpec((BLK, HID), lambda i: (i, 0)),
        full((HID, HID)), full((1, HID)),
        full((HID, HID)), full((1, HID)),
        full((HID, HID)), full((1, HID)),
    ]
    args = [m, mf, W_o1, b_o1, W_o2, b_o2, W_o3, b_o3]
    kwargs = {}
    if mout_prev is not None:
        in_specs.append(pl.BlockSpec(memory_space=pltpu.MemorySpace.HBM))
        args.append(mout_prev)
        kwargs["input_output_aliases"] = {8: 0}
    return pl.pallas_call(
        body,
        grid=(nblk,),
        in_specs=in_specs,
        out_specs=pl.BlockSpec((BLK, HID), lambda i: (i + blk0, 0)),
        out_shape=jax.ShapeDtypeStruct((E, HID), jnp.float32),
        **kwargs,
    )(*args)


def _tc_nodes(h, agg2, agg2b, Wu1h, Wu1a, b_u1, W_u2, b_u2):
    BLK = 1000

    def body(h_ref, agg_ref, aggb_ref, wa, wb, bu1, wu2, bu2, hout_ref):
        hh = h_ref[...]
        agg = (agg_ref[0] + agg_ref[1]) + (aggb_ref[0] + aggb_ref[1])
        t = jax.nn.silu(
            jnp.dot(hh, wa[...], preferred_element_type=jnp.float32)
            + jnp.dot(agg, wb[...], preferred_element_type=jnp.float32)
            + bu1[...])
        hout_ref[...] = hh + jnp.dot(t, wu2[...],
                                     preferred_element_type=jnp.float32) + bu2[...]

    def full(shape):
        return pl.BlockSpec(shape, lambda i: tuple(0 for _ in shape))

    return pl.pallas_call(
        body,
        grid=(N // BLK,),
        in_specs=[
            pl.BlockSpec((BLK, HID), lambda i: (i, 0)),
            pl.BlockSpec((NC, BLK, HID), lambda i: (0, i, 0)),
            pl.BlockSpec((NC, BLK, HID), lambda i: (0, i, 0)),
            full((HID, HID)), full((HID, HID)), full((1, HID)),
            full((HID, HID)), full((1, HID)),
        ],
        out_specs=pl.BlockSpec((BLK, HID), lambda i: (i, 0)),
        out_shape=jax.ShapeDtypeStruct((N, HID), jnp.float32),
    )(h, agg2, agg2b, Wu1h, Wu1a, b_u1, W_u2, b_u2)


def kernel(h, m, radial_basis, spherical_basis, edge_index, triplets,
           W_r1, b_r1, W_r2, b_r2, W_sp, b_sp, W_u1, b_u1, W_u2, b_u2,
           W_o1, b_o1, W_o2, b_o2, W_o3, b_o3):
    idx_ji = triplets[:, 0].astype(jnp.int32)
    idx_kj = triplets[:, 1].astype(jnp.int32)
    dst = edge_index[1].astype(jnp.int32)

    sbT8 = jnp.concatenate(
        [spherical_basis.T, jnp.ones((1, E), jnp.float32)], axis=0)
    sbT8 = jnp.pad(sbT8, ((0, 0), (0, EAW - E)))
    w8 = jnp.concatenate([jnp.mean(W_sp, axis=1), jnp.mean(b_sp)[None]])
    w8b = jnp.broadcast_to(w8[:, None], (8, 4096))

    sigma = []
    for q in range(4):
        for i in range(16):
            sigma += [q * 32 + i, q * 32 + 16 + i]
    Pfwd = jnp.array(sigma, jnp.int32)

    rf = _tc_rf(radial_basis.T, W_r1, b_r1.reshape(1, HID), W_r2[:, Pfwd],
                b_r2[Pfwd].reshape(1, HID))
    aw = _tc_aw(sbT8, w8b)

    NCH1 = 2688
    bo1, bo2, bo3 = (b_o1.reshape(1, HID), b_o2.reshape(1, HID),
                     b_o3.reshape(1, HID))
    Wo1p, Wo2p, Wo3p = W_o1[Pfwd, :], W_o2[Pfwd, :], W_o3[Pfwd, :]
    mf1, agg2a = _sc_mega(m, rf, aw, idx_kj, idx_ji, dst, 0, NCH1)
    mf2, agg2b = _sc_mega(m, rf, aw, idx_kj, idx_ji, dst, NCH1, NCH - NCH1)
    mo1 = _tc_edges(m, mf1, Wo1p, bo1, Wo2p, bo2, Wo3p, bo3, 0, None)
    m_out = _tc_edges(m, mf2, Wo1p, bo1, Wo2p, bo2, Wo3p, bo3,
                      NCH1 * CHUNK // 2560, mo1)
    h_out = _tc_nodes(h, agg2a, agg2b, W_u1[:HID], W_u1[HID:],
                      b_u1.reshape(1, HID), W_u2, b_u2.reshape(1, HID))
    return (h_out, m_out)


# speedup vs baseline: 1.6433x; 1.6433x over previous
import functools

import jax
import jax.numpy as jnp
from jax import lax
from jax.experimental import pallas as pl
from jax.experimental.pallas import tpu as pltpu
from jax.experimental.pallas import tpu_sc as plsc

E = 320000
N = 10000
HID = 128
NRAD = 16
CHUNK = 80
NCH = E // CHUNK
NC = 2
NS = 16
NW = NC * NS
NDRAIN = N // CHUNK
L = 16


def _tc_rf(rbT, W_r1, b_r1, W_r2, b_r2):
    BLK = 2560

    def body(rbT_ref, wr1, br1, wr2, br2, rf_ref):
        x = lax.dot_general(rbT_ref[...], wr1[...],
                            (((0,), (0,)), ((), ())),
                            preferred_element_type=jnp.float32)
        t = jax.nn.silu(x + br1[...])
        rf_ref[...] = jnp.dot(t, wr2[...],
                              preferred_element_type=jnp.float32) + br2[...]

    def full(shape):
        return pl.BlockSpec(shape, lambda i: tuple(0 for _ in shape))

    return pl.pallas_call(
        body,
        grid=(E // BLK,),
        in_specs=[
            pl.BlockSpec((NRAD, BLK), lambda i: (0, i)),
            full((NRAD, HID)), full((1, HID)), full((HID, HID)), full((1, HID)),
        ],
        out_specs=pl.BlockSpec((BLK, HID), lambda i: (i, 0)),
        out_shape=jax.ShapeDtypeStruct((E, HID), jnp.float32),
    )(rbT, W_r1, b_r1, W_r2, b_r2)


EAW = 327680


def _tc_aw(sbT8, w8b):
    BLK = 4096

    def body(sb_ref, w8_ref, aw_ref):
        z = jnp.sum(sb_ref[...] * w8_ref[...], axis=0)
        aw_ref[...] = jax.nn.sigmoid(z)

    return pl.pallas_call(
        body,
        grid=(EAW // BLK,),
        in_specs=[
            pl.BlockSpec((8, BLK), lambda i: (0, i)),
            pl.BlockSpec((8, BLK), lambda i: (0, 0)),
        ],
        out_specs=pl.BlockSpec((BLK,), lambda i: (i,)),
        out_shape=jax.ShapeDtypeStruct((EAW,), jnp.float32),
    )(sbT8, w8b)


NITER = NCH // NW

_DN = lax.GatherDimensionNumbers(
    offset_dims=(), collapsed_slice_dims=(0,), start_index_map=(0,))


def _sc_mega(m, rf, aw, ikj, iji, dst, chunk0, nchunks):
    mesh = plsc.VectorSubcoreMesh(core_axis_name="c", subcore_axis_name="s")
    niter = nchunks // NW
    goff = chunk0 * CHUNK

    nbuf = 2
    scratch = []
    for _ in range(nbuf):
        scratch += [
            pltpu.VMEM((CHUNK,), jnp.int32),
            pltpu.VMEM((CHUNK,), jnp.int32),
            pltpu.VMEM((CHUNK,), jnp.int32),
            pltpu.VMEM((CHUNK, HID), jnp.float32),
            pltpu.VMEM((CHUNK, HID), jnp.float32),
            pltpu.VMEM((CHUNK,), jnp.float32),
            pltpu.SemaphoreType.DMA,
            pltpu.SemaphoreType.DMA,
        ]
    scratch.append(pltpu.VMEM_SHARED((N, HID), jnp.float32))

    @functools.partial(
        pl.kernel,
        out_type=(
            jax.ShapeDtypeStruct((nchunks * CHUNK, HID), jnp.float32),
            jax.ShapeDtypeStruct((NC, N, HID), jnp.float32),
        ),
        mesh=mesh,
        scratch_types=scratch,
        compiler_params=pltpu.CompilerParams(use_tc_tiling_on_sc=False),
    )
    def k(m_hbm, rf_hbm, aw_hbm, ikj_hbm, iji_hbm, dst_hbm,
          mf_out, agg_out, *bufs):
        sets = []
        for b in range(nbuf):
            sets.append(bufs[b * 8:(b + 1) * 8])
        agg_sh = bufs[nbuf * 8]
        c = lax.axis_index("c")
        s = lax.axis_index("s")
        wid = s * NC + c

        zb = sets[0][3]

        def zrow(i, carry):
            r = i // (HID // L)
            q = i % (HID // L)
            zb[r, pl.ds(q * L, L)] = jnp.zeros((L,), jnp.float32)
            return carry

        lax.fori_loop(0, CHUNK * (HID // L), zrow, 0)
        for j in range((NDRAIN + NS - 1) // NS):
            t = j * NS + s

            @pl.when(t < NDRAIN)
            def _():
                pltpu.sync_copy(
                    zb, agg_sh.at[pl.ds(pl.multiple_of(t * CHUNK, CHUNK),
                                        CHUNK)])

        plsc.subcore_barrier()

        def issue_idx(i, st):
            base = pl.multiple_of(goff + (i * NW + wid) * CHUNK, CHUNK)
            a = pltpu.async_copy(ikj_hbm.at[pl.ds(base, CHUNK)], st[0], st[6])
            b = pltpu.async_copy(iji_hbm.at[pl.ds(base, CHUNK)], st[1], st[6])
            d = pltpu.async_copy(dst_hbm.at[pl.ds(base, CHUNK)], st[2], st[6])
            return a, b, d

        def issue_idx_drain(st):
            pltpu.make_async_copy(ikj_hbm.at[pl.ds(0, CHUNK)], st[0], st[6]).wait()
            pltpu.make_async_copy(iji_hbm.at[pl.ds(0, CHUNK)], st[1], st[6]).wait()
            pltpu.make_async_copy(dst_hbm.at[pl.ds(0, CHUNK)], st[2], st[6]).wait()

        def issue_gath(st):
            pltpu.async_copy(m_hbm.at[st[0]], st[3], st[7])
            pltpu.async_copy(rf_hbm.at[st[0]], st[4], st[7])
            pltpu.async_copy(aw_hbm.at[st[1]], st[5], st[7])

        def wait_gath(st):
            pltpu.make_async_copy(m_hbm.at[st[0]], st[3], st[7]).wait()
            pltpu.make_async_copy(rf_hbm.at[st[0]], st[4], st[7]).wait()
            pltpu.make_async_copy(aw_hbm.at[st[1]], st[5], st[7]).wait()

        def mul(st):
            mbuf, rfbuf, awbuf = st[3], st[4], st[5]

            def mul_group(p, carry2):
                aw16 = awbuf[pl.ds(pl.multiple_of(p * L, L), L)]
                for l in range(L):
                    awr = lax.gather(
                        aw16, jnp.full((L, 1), l, jnp.int32), _DN,
                        slice_sizes=(1,),
                        mode=lax.GatherScatterMode.PROMISE_IN_BOUNDS)
                    r = p * L + l
                    for g in range(HID // L):
                        sl = pl.ds(g * L, L)
                        mbuf[r, sl] = mbuf[r, sl] * rfbuf[r, sl] * awr
                return carry2

            lax.fori_loop(0, CHUNK // L, mul_group, 0)

        def consume(i, st):
            wait_gath(st)
            mul(st)
            base = pl.multiple_of((i * NW + wid) * CHUNK, CHUNK)
            pltpu.sync_copy(st[3], mf_out.at[pl.ds(base, CHUNK)])
            pltpu.sync_copy(st[3], agg_sh.at[st[2]], add=True)

        NITER = niter

        A, B = sets[0], sets[1]
        issue_idx(0, A)
        issue_idx_drain(A)
        issue_gath(A)
        issue_idx(1, B)

        def pair_body(p, carry):
            i0 = p * 2
            i1 = i0 + 1

            @pl.when(i1 < NITER)
            def _():
                issue_idx_drain(B)
                issue_gath(B)

            consume(i0, A)

            @pl.when(i0 + 2 < NITER)
            def _():
                issue_idx(i0 + 2, A)

            @pl.when(i1 < NITER)
            def _():
                @pl.when(i1 + 1 < NITER)
                def _():
                    issue_idx_drain(A)
                    issue_gath(A)

                consume(i1, B)

                @pl.when(i1 + 2 < NITER)
                def _():
                    issue_idx(i1 + 2, B)

            return carry

        lax.fori_loop(0, (NITER + 1) // 2, pair_body, 0)

        plsc.subcore_barrier()

        for j in range((NDRAIN + NS - 1) // NS):
            t = j * NS + s

            @pl.when(t < NDRAIN)
            def _():
                base = pl.multiple_of(t * CHUNK, CHUNK)
                pltpu.sync_copy(agg_sh.at[pl.ds(base, CHUNK)],
                                agg_out.at[c, pl.ds(base, CHUNK)])

    return k(m, rf, aw, ikj, iji, dst)


def _tc_edges(m, mf, W_o1, b_o1, W_o2, b_o2, W_o3, b_o3, blk0, mout_prev):
    BLK = 2560
    nblk = mf.shape[0] // BLK

    def body(*refs):
        m_ref, mf_ref, wo1, bo1, wo2, bo2, wo3, bo3 = refs[:8]
        mout_ref = refs[-1]
        silu = jax.nn.silu
        mfv = mf_ref[...]
        mn = silu(jnp.dot(mfv, wo1[...], preferred_element_type=jnp.float32)
                  + bo1[...])
        mn = mn + silu(jnp.dot(mfv, wo2[...], preferred_element_type=jnp.float32)
                       + bo2[...])
        mn = mn + silu(jnp.dot(mfv, wo3[...], preferred_element_type=jnp.float32)
                       + bo3[...])
        mout_ref[...] = m_ref[...] + mn

    def full(shape):
        return pl.BlockSpec(shape, lambda i: tuple(0 for _ in shape))

    in_specs = [
        pl.BlockSpec((BLK, HID), lambda i: (i + blk0, 0)),
        pl.BlockSpec((BLK, HID), lambda i: (i, 0)),
        full((HID, HID)), full((1, HID)),
        full((HID, HID)), full((1, HID)),
        full((HID, HID)), full((1, HID)),
    ]
    args = [m, mf, W_o1, b_o1, W_o2, b_o2, W_o3, b_o3]
    kwargs = {}
    if mout_prev is not None:
        in_specs.append(pl.BlockSpec(memory_space=pltpu.MemorySpace.HBM))
        args.append(mout_prev)
        kwargs["input_output_aliases"] = {8: 0}
    return pl.pallas_call(
        body,
        grid=(nblk,),
        in_specs=in_specs,
        out_specs=pl.BlockSpec((BLK, HID), lambda i: (i + blk0, 0)),
        out_shape=jax.ShapeDtypeStruct((E, HID), jnp.float32),
        **kwargs,
    )(*args)


def _tc_nodes(h, agg2, agg2b, Wu1h, Wu1a, b_u1, W_u2, b_u2):
    BLK = 1000

    def body(h_ref, agg_ref, aggb_ref, wa, wb, bu1, wu2, bu2, hout_ref):
        hh = h_ref[...]
        agg = (agg_ref[0] + agg_ref[1]) + (aggb_ref[0] + aggb_ref[1])
        t = jax.nn.silu(
            jnp.dot(hh, wa[...], preferred_element_type=jnp.float32)
            + jnp.dot(agg, wb[...], preferred_element_type=jnp.float32)
            + bu1[...])
        hout_ref[...] = hh + jnp.dot(t, wu2[...],
                                     preferred_element_type=jnp.float32) + bu2[...]

    def full(shape):
        return pl.BlockSpec(shape, lambda i: tuple(0 for _ in shape))

    return pl.pallas_call(
        body,
        grid=(N // BLK,),
        in_specs=[
            pl.BlockSpec((BLK, HID), lambda i: (i, 0)),
            pl.BlockSpec((NC, BLK, HID), lambda i: (0, i, 0)),
            pl.BlockSpec((NC, BLK, HID), lambda i: (0, i, 0)),
            full((HID, HID)), full((HID, HID)), full((1, HID)),
            full((HID, HID)), full((1, HID)),
        ],
        out_specs=pl.BlockSpec((BLK, HID), lambda i: (i, 0)),
        out_shape=jax.ShapeDtypeStruct((N, HID), jnp.float32),
    )(h, agg2, agg2b, Wu1h, Wu1a, b_u1, W_u2, b_u2)


def kernel(h, m, radial_basis, spherical_basis, edge_index, triplets,
           W_r1, b_r1, W_r2, b_r2, W_sp, b_sp, W_u1, b_u1, W_u2, b_u2,
           W_o1, b_o1, W_o2, b_o2, W_o3, b_o3):
    idx_ji = triplets[:, 0].astype(jnp.int32)
    idx_kj = triplets[:, 1].astype(jnp.int32)
    dst = edge_index[1].astype(jnp.int32)

    sbT8 = jnp.concatenate(
        [spherical_basis.T, jnp.ones((1, E), jnp.float32)], axis=0)
    sbT8 = jnp.pad(sbT8, ((0, 0), (0, EAW - E)))
    w8 = jnp.concatenate([jnp.mean(W_sp, axis=1), jnp.mean(b_sp)[None]])
    w8b = jnp.broadcast_to(w8[:, None], (8, 4096))

    rf = _tc_rf(radial_basis.T, W_r1, b_r1.reshape(1, HID), W_r2,
                b_r2.reshape(1, HID))
    aw = _tc_aw(sbT8, w8b)

    NCH1 = 2688
    bo1, bo2, bo3 = (b_o1.reshape(1, HID), b_o2.reshape(1, HID),
                     b_o3.reshape(1, HID))
    mf1, agg2a = _sc_mega(m, rf, aw, idx_kj, idx_ji, dst, 0, NCH1)
    mf2, agg2b = _sc_mega(m, rf, aw, idx_kj, idx_ji, dst, NCH1, NCH - NCH1)
    mo1 = _tc_edges(m, mf1, W_o1, bo1, W_o2, bo2, W_o3, bo3, 0, None)
    m_out = _tc_edges(m, mf2, W_o1, bo1, W_o2, bo2, W_o3, bo3,
                      NCH1 * CHUNK // 2560, mo1)
    h_out = _tc_nodes(h, agg2a, agg2b, W_u1[:HID], W_u1[HID:],
                      b_u1.reshape(1, HID), W_u2, b_u2.reshape(1, HID))
    return (h_out, m_out)


# speedup vs baseline: 1.7750x; 1.0801x over previous
import functools

import jax
import jax.numpy as jnp
from jax import lax
from jax.experimental import pallas as pl
from jax.experimental.pallas import tpu as pltpu
from jax.experimental.pallas import tpu_sc as plsc

E = 320000
N = 10000
HID = 128
NRAD = 16
CHUNK = 80
NCH = E // CHUNK
NC = 2
NS = 16
NW = NC * NS
NDRAIN = N // CHUNK
L = 16


def _tc_rf(rbT, W_r1, b_r1, W_r2, b_r2):
    BLK = 6400

    def body(rbT_ref, wr1, br1, wr2, br2, rf_ref):
        x = lax.dot_general(rbT_ref[...], wr1[...],
                            (((0,), (0,)), ((), ())),
                            preferred_element_type=jnp.float32)
        t = jax.nn.silu(x + br1[...])
        rf_ref[...] = jnp.dot(t, wr2[...],
                              preferred_element_type=jnp.float32) + br2[...]

    def full(shape):
        return pl.BlockSpec(shape, lambda i: tuple(0 for _ in shape))

    return pl.pallas_call(
        body,
        grid=(E // BLK,),
        in_specs=[
            pl.BlockSpec((NRAD, BLK), lambda i: (0, i)),
            full((NRAD, HID)), full((1, HID)), full((HID, HID)), full((1, HID)),
        ],
        out_specs=pl.BlockSpec((BLK, HID), lambda i: (i, 0)),
        out_shape=jax.ShapeDtypeStruct((E, HID), jnp.float32),
    )(rbT, W_r1, b_r1, W_r2, b_r2)


EAW = 327680


def _tc_aw(sbT8, w8b):
    BLK = 8192

    def body(sb_ref, w8_ref, aw_ref):
        z = jnp.sum(sb_ref[...] * w8_ref[...], axis=0)
        aw_ref[...] = jax.nn.sigmoid(z)

    return pl.pallas_call(
        body,
        grid=(EAW // BLK,),
        in_specs=[
            pl.BlockSpec((8, BLK), lambda i: (0, i)),
            pl.BlockSpec((8, BLK), lambda i: (0, 0)),
        ],
        out_specs=pl.BlockSpec((BLK,), lambda i: (i,)),
        out_shape=jax.ShapeDtypeStruct((EAW,), jnp.float32),
    )(sbT8, w8b)


NITER = NCH // NW

_DN = lax.GatherDimensionNumbers(
    offset_dims=(), collapsed_slice_dims=(0,), start_index_map=(0,))


def _sc_mega(m, rf, aw, ikj, iji, dst, chunk0, nchunks):
    mesh = plsc.VectorSubcoreMesh(core_axis_name="c", subcore_axis_name="s")
    niter = nchunks // NW
    goff = chunk0 * CHUNK

    nbuf = 2
    scratch = []
    for _ in range(nbuf):
        scratch += [
            pltpu.VMEM((CHUNK,), jnp.int32),
            pltpu.VMEM((CHUNK,), jnp.int32),
            pltpu.VMEM((CHUNK,), jnp.int32),
            pltpu.VMEM((CHUNK, HID), jnp.float32),
            pltpu.VMEM((CHUNK, HID), jnp.float32),
            pltpu.VMEM((CHUNK,), jnp.float32),
            pltpu.SemaphoreType.DMA,
            pltpu.SemaphoreType.DMA,
        ]
    scratch.append(pltpu.VMEM_SHARED((N, HID), jnp.float32))

    @functools.partial(
        pl.kernel,
        out_type=(
            jax.ShapeDtypeStruct((nchunks * CHUNK, HID), jnp.float32),
            jax.ShapeDtypeStruct((NC, N, HID), jnp.float32),
        ),
        mesh=mesh,
        scratch_types=scratch,
        compiler_params=pltpu.CompilerParams(use_tc_tiling_on_sc=False),
    )
    def k(m_hbm, rf_hbm, aw_hbm, ikj_hbm, iji_hbm, dst_hbm,
          mf_out, agg_out, *bufs):
        sets = []
        for b in range(nbuf):
            sets.append(bufs[b * 8:(b + 1) * 8])
        agg_sh = bufs[nbuf * 8]
        c = lax.axis_index("c")
        s = lax.axis_index("s")
        wid = s * NC + c

        zb = sets[0][3]

        def zrow(i, carry):
            r = i // (HID // L)
            q = i % (HID // L)
            zb[r, pl.ds(q * L, L)] = jnp.zeros((L,), jnp.float32)
            return carry

        lax.fori_loop(0, CHUNK * (HID // L), zrow, 0)
        for j in range((NDRAIN + NS - 1) // NS):
            t = j * NS + s

            @pl.when(t < NDRAIN)
            def _():
                pltpu.sync_copy(
                    zb, agg_sh.at[pl.ds(pl.multiple_of(t * CHUNK, CHUNK),
                                        CHUNK)])

        plsc.subcore_barrier()

        def issue_idx(i, st):
            base = pl.multiple_of(goff + (i * NW + wid) * CHUNK, CHUNK)
            a = pltpu.async_copy(ikj_hbm.at[pl.ds(base, CHUNK)], st[0], st[6])
            b = pltpu.async_copy(iji_hbm.at[pl.ds(base, CHUNK)], st[1], st[6])
            d = pltpu.async_copy(dst_hbm.at[pl.ds(base, CHUNK)], st[2], st[6])
            return a, b, d

        def issue_idx_drain(st):
            pltpu.make_async_copy(ikj_hbm.at[pl.ds(0, CHUNK)], st[0], st[6]).wait()
            pltpu.make_async_copy(iji_hbm.at[pl.ds(0, CHUNK)], st[1], st[6]).wait()
            pltpu.make_async_copy(dst_hbm.at[pl.ds(0, CHUNK)], st[2], st[6]).wait()

        def issue_gath(st):
            pltpu.async_copy(m_hbm.at[st[0]], st[3], st[7])
            pltpu.async_copy(rf_hbm.at[st[0]], st[4], st[7])
            pltpu.async_copy(aw_hbm.at[st[1]], st[5], st[7])

        def wait_gath(st):
            pltpu.make_async_copy(m_hbm.at[st[0]], st[3], st[7]).wait()
            pltpu.make_async_copy(rf_hbm.at[st[0]], st[4], st[7]).wait()
            pltpu.make_async_copy(aw_hbm.at[st[1]], st[5], st[7]).wait()

        def mul(st):
            mbuf, rfbuf, awbuf = st[3], st[4], st[5]

            def mul_group(p, carry2):
                aw16 = awbuf[pl.ds(pl.multiple_of(p * L, L), L)]
                for l in range(L):
                    awr = lax.gather(
                        aw16, jnp.full((L, 1), l, jnp.int32), _DN,
                        slice_sizes=(1,),
                        mode=lax.GatherScatterMode.PROMISE_IN_BOUNDS)
                    r = p * L + l
                    for g in range(HID // L):
                        sl = pl.ds(g * L, L)
                        mbuf[r, sl] = mbuf[r, sl] * rfbuf[r, sl] * awr
                return carry2

            lax.fori_loop(0, CHUNK // L, mul_group, 0)

        def consume(i, st):
            wait_gath(st)
            mul(st)
            base = pl.multiple_of((i * NW + wid) * CHUNK, CHUNK)
            pltpu.sync_copy(st[3], mf_out.at[pl.ds(base, CHUNK)])
            pltpu.sync_copy(st[3], agg_sh.at[st[2]], add=True)

        NITER = niter

        A, B = sets[0], sets[1]
        issue_idx(0, A)
        issue_idx_drain(A)
        issue_gath(A)
        issue_idx(1, B)

        def pair_body(p, carry):
            i0 = p * 2
            i1 = i0 + 1

            @pl.when(i1 < NITER)
            def _():
                issue_idx_drain(B)
                issue_gath(B)

            consume(i0, A)

            @pl.when(i0 + 2 < NITER)
            def _():
                issue_idx(i0 + 2, A)

            @pl.when(i1 < NITER)
            def _():
                @pl.when(i1 + 1 < NITER)
                def _():
                    issue_idx_drain(A)
                    issue_gath(A)

                consume(i1, B)

                @pl.when(i1 + 2 < NITER)
                def _():
                    issue_idx(i1 + 2, B)

            return carry

        lax.fori_loop(0, (NITER + 1) // 2, pair_body, 0)

        plsc.subcore_barrier()

        for j in range((NDRAIN + NS - 1) // NS):
            t = j * NS + s

            @pl.when(t < NDRAIN)
            def _():
                base = pl.multiple_of(t * CHUNK, CHUNK)
                pltpu.sync_copy(agg_sh.at[pl.ds(base, CHUNK)],
                                agg_out.at[c, pl.ds(base, CHUNK)])

    return k(m, rf, aw, ikj, iji, dst)


def _tc_edges(m, mf, W_o1, b_o1, W_o2, b_o2, W_o3, b_o3, blk0, mout_prev):
    BLK = 2560
    nblk = mf.shape[0] // BLK

    def body(*refs):
        m_ref, mf_ref, wo1, bo1, wo2, bo2, wo3, bo3 = refs[:8]
        mout_ref = refs[-1]
        silu = jax.nn.silu
        mfv = mf_ref[...]
        mn = silu(jnp.dot(mfv, wo1[...], preferred_element_type=jnp.float32)
                  + bo1[...])
        mn = mn + silu(jnp.dot(mfv, wo2[...], preferred_element_type=jnp.float32)
                       + bo2[...])
        mn = mn + silu(jnp.dot(mfv, wo3[...], preferred_element_type=jnp.float32)
                       + bo3[...])
        mout_ref[...] = m_ref[...] + mn

    def full(shape):
        return pl.BlockSpec(shape, lambda i: tuple(0 for _ in shape))

    in_specs = [
        pl.BlockSpec((BLK, HID), lambda i: (i + blk0, 0)),
        pl.BlockSpec((BLK, HID), lambda i: (i, 0)),
        full((HID, HID)), full((1, HID)),
        full((HID, HID)), full((1, HID)),
        full((HID, HID)), full((1, HID)),
    ]
    args = [m, mf, W_o1, b_o1, W_o2, b_o2, W_o3, b_o3]
    kwargs = {}
    if mout_prev is not None:
        in_specs.append(pl.BlockSpec(memory_space=pltpu.MemorySpace.HBM))
        args.append(mout_prev)
        kwargs["input_output_aliases"] = {8: 0}
    return pl.pallas_call(
        body,
        grid=(nblk,),
        in_specs=in_specs,
        out_specs=pl.BlockSpec((BLK, HID), lambda i: (i + blk0, 0)),
        out_shape=jax.ShapeDtypeStruct((E, HID), jnp.float32),
        **kwargs,
    )(*args)


def _tc_nodes(h, agg2, agg2b, Wu1h, Wu1a, b_u1, W_u2, b_u2):
    BLK = 1000

    def body(h_ref, agg_ref, aggb_ref, wa, wb, bu1, wu2, bu2, hout_ref):
        hh = h_ref[...]
        agg = (agg_ref[0] + agg_ref[1]) + (aggb_ref[0] + aggb_ref[1])
        t = jax.nn.silu(
            jnp.dot(hh, wa[...], preferred_element_type=jnp.float32)
            + jnp.dot(agg, wb[...], preferred_element_type=jnp.float32)
            + bu1[...])
        hout_ref[...] = hh + jnp.dot(t, wu2[...],
                                     preferred_element_type=jnp.float32) + bu2[...]

    def full(shape):
        return pl.BlockSpec(shape, lambda i: tuple(0 for _ in shape))

    return pl.pallas_call(
        body,
        grid=(N // BLK,),
        in_specs=[
            pl.BlockSpec((BLK, HID), lambda i: (i, 0)),
            pl.BlockSpec((NC, BLK, HID), lambda i: (0, i, 0)),
            pl.BlockSpec((NC, BLK, HID), lambda i: (0, i, 0)),
            full((HID, HID)), full((HID, HID)), full((1, HID)),
            full((HID, HID)), full((1, HID)),
        ],
        out_specs=pl.BlockSpec((BLK, HID), lambda i: (i, 0)),
        out_shape=jax.ShapeDtypeStruct((N, HID), jnp.float32),
    )(h, agg2, agg2b, Wu1h, Wu1a, b_u1, W_u2, b_u2)


def kernel(h, m, radial_basis, spherical_basis, edge_index, triplets,
           W_r1, b_r1, W_r2, b_r2, W_sp, b_sp, W_u1, b_u1, W_u2, b_u2,
           W_o1, b_o1, W_o2, b_o2, W_o3, b_o3):
    idx_ji = triplets[:, 0].astype(jnp.int32)
    idx_kj = triplets[:, 1].astype(jnp.int32)
    dst = edge_index[1].astype(jnp.int32)

    sbT8 = jnp.concatenate(
        [spherical_basis.T, jnp.ones((1, E), jnp.float32)], axis=0)
    sbT8 = jnp.pad(sbT8, ((0, 0), (0, EAW - E)))
    w8 = jnp.concatenate([jnp.mean(W_sp, axis=1), jnp.mean(b_sp)[None]])
    w8b = jnp.broadcast_to(w8[:, None], (8, 8192))

    rf = _tc_rf(radial_basis.T, W_r1, b_r1.reshape(1, HID), W_r2,
                b_r2.reshape(1, HID))
    aw = _tc_aw(sbT8, w8b)

    NCH1 = 2688
    bo1, bo2, bo3 = (b_o1.reshape(1, HID), b_o2.reshape(1, HID),
                     b_o3.reshape(1, HID))
    mf1, agg2a = _sc_mega(m, rf, aw, idx_kj, idx_ji, dst, 0, NCH1)
    mf2, agg2b = _sc_mega(m, rf, aw, idx_kj, idx_ji, dst, NCH1, NCH - NCH1)
    mo1 = _tc_edges(m, mf1, W_o1, bo1, W_o2, bo2, W_o3, bo3, 0, None)
    m_out = _tc_edges(m, mf2, W_o1, bo1, W_o2, bo2, W_o3, bo3,
                      NCH1 * CHUNK // 2560, mo1)
    h_out = _tc_nodes(h, agg2a, agg2b, W_u1[:HID], W_u1[HID:],
                      b_u1.reshape(1, HID), W_u2, b_u2.reshape(1, HID))
    return (h_out, m_out)
